# baseline (speedup 1.0000x reference)
import functools

import jax
import jax.numpy as jnp
import numpy as np
from jax import lax
from jax.experimental import pallas as pl
from jax.experimental.pallas import tpu as pltpu
from jax.experimental.pallas import tpu_sc as plsc

BATCH = 4096
D = 128
L = 16


def _build():
    info = plsc.get_sparse_core_info()
    nc, ns = info.num_cores, info.num_subcores
    nw = nc * ns
    bpw = BATCH // nw
    mesh = plsc.VectorSubcoreMesh(core_axis_name="c", subcore_axis_name="s")

    @functools.partial(
        pl.kernel,
        mesh=mesh,
        compiler_params=pltpu.CompilerParams(needs_layout_passes=False),
        out_type=jax.ShapeDtypeStruct((BATCH,), jnp.float32),
        scratch_types=[
            pltpu.VMEM((bpw,), jnp.int32),
            pltpu.VMEM((bpw,), jnp.int32),
            pltpu.VMEM((bpw, D), jnp.float32),
            pltpu.VMEM((bpw, D), jnp.float32),
            pltpu.VMEM((bpw,), jnp.float32),
            [pltpu.SemaphoreType.DMA] * (bpw // (2 * L)),
        ],
    )
    def scores_kernel(uid_hbm, iid_hbm, ut_hbm, it_hbm, out_hbm,
                      uidx_v, iidx_v, urows_v, irows_v, out_v, sems):
        wid = lax.axis_index("s") * nc + lax.axis_index("c")
        base = wid * bpw
        rpb = 2 * L
        nb = bpw // rpb
        pltpu.sync_copy(uid_hbm.at[pl.ds(base, bpw)], uidx_v)
        pltpu.sync_copy(iid_hbm.at[pl.ds(base, bpw)], iidx_v)
        copies = []
        for k in range(nb):
            sl = pl.ds(k * rpb, rpb)
            cu = pltpu.async_copy(ut_hbm.at[uidx_v.at[sl]], urows_v.at[sl], sems[k])
            ci = pltpu.async_copy(it_hbm.at[iidx_v.at[sl]], irows_v.at[sl], sems[k])
            copies.append((cu, ci))

        lanes = lax.iota(jnp.int32, L)
        masks = [(lanes & (1 << t)) != 0 for t in range(4)]

        def rot(v, k):
            return jnp.concatenate([v[k:], v[:k]])

        keys = [plsc.bitcast(lanes ^ (1 << t), jnp.uint32) for t in range(4)]

        def group(g):
            accs = []
            for j in range(L):
                b = g * L + j
                ts = [urows_v[b, pl.ds(c * L, L)] * irows_v[b, pl.ds(c * L, L)]
                      for c in range(D // L)]
                while len(ts) > 1:
                    ts = [ts[2 * i] + ts[2 * i + 1] for i in range(len(ts) // 2)]
                accs.append(ts[0])
            vecs = accs
            for t in range(4):
                m, key = masks[t], keys[t]
                nxt = []
                for p in range(len(vecs) // 2):
                    a, b2 = vecs[2 * p], vecs[2 * p + 1]
                    x = jnp.where(m, b2, a)
                    y = jnp.where(m, a, b2)
                    _, yx = plsc.sort_key_val(key, y)
                    nxt.append(x + yx)
                vecs = nxt
            out_v[pl.ds(g * L, L)] = vecs[0]

        def block(k, carry):
            for kk in range(nb):
                @pl.when(k == kk)
                def _(kk=kk):
                    copies[kk][0].wait()
                    copies[kk][1].wait()
            group(2 * k)
            group(2 * k + 1)
            return carry

        lax.fori_loop(0, nb, block, 0)
        pltpu.sync_copy(out_v, out_hbm.at[pl.ds(base, bpw)])

    return scores_kernel


_scores = _build()


def kernel(user_id, user_features, item_id, item_features, position,
           user_table, item_table):
    del user_features, item_features, position
    return _scores(user_id, item_id, user_table, item_table)


# speedup vs baseline: 1.0570x; 1.0570x over previous
import functools

import jax
import jax.numpy as jnp
import numpy as np
from jax import lax
from jax.experimental import pallas as pl
from jax.experimental.pallas import tpu as pltpu
from jax.experimental.pallas import tpu_sc as plsc

BATCH = 4096
D = 128
L = 16


def _build():
    info = plsc.get_sparse_core_info()
    nc, ns = info.num_cores, info.num_subcores
    nw = nc * ns
    bpw = BATCH // nw
    mesh = plsc.VectorSubcoreMesh(core_axis_name="c", subcore_axis_name="s")

    @functools.partial(
        pl.kernel,
        mesh=mesh,
        compiler_params=pltpu.CompilerParams(
            needs_layout_passes=False,
            disable_bounds_checks=True,
            disable_semaphore_checks=True,
            skip_device_barrier=True,
        ),
        out_type=jax.ShapeDtypeStruct((BATCH,), jnp.float32),
        scratch_types=[
            pltpu.VMEM((bpw,), jnp.int32),
            pltpu.VMEM((bpw,), jnp.int32),
            pltpu.VMEM((bpw, D), jnp.float32),
            pltpu.VMEM((bpw, D), jnp.float32),
            pltpu.VMEM((bpw,), jnp.float32),
            [pltpu.SemaphoreType.DMA] * (bpw // (2 * L) + 1),
        ],
    )
    def scores_kernel(uid_hbm, iid_hbm, ut_hbm, it_hbm, out_hbm,
                      uidx_v, iidx_v, urows_v, irows_v, out_v, sems):
        wid = lax.axis_index("s") * nc + lax.axis_index("c")
        base = wid * bpw
        rpb = 2 * L
        nb = bpw // rpb
        cid0 = pltpu.async_copy(uid_hbm.at[pl.ds(base, bpw)], uidx_v, sems[nb])
        cid1 = pltpu.async_copy(iid_hbm.at[pl.ds(base, bpw)], iidx_v, sems[nb])
        cid0.wait()
        cid1.wait()
        copies = []
        for k in range(nb):
            sl = pl.ds(k * rpb, rpb)
            cu = pltpu.async_copy(ut_hbm.at[uidx_v.at[sl]], urows_v.at[sl], sems[k])
            ci = pltpu.async_copy(it_hbm.at[iidx_v.at[sl]], irows_v.at[sl], sems[k])
            copies.append((cu, ci))

        lanes = lax.iota(jnp.int32, L)
        masks = [(lanes & (1 << t)) != 0 for t in range(4)]

        def rot(v, k):
            return jnp.concatenate([v[k:], v[:k]])

        keys = [plsc.bitcast(lanes ^ (1 << t), jnp.uint32) for t in range(4)]

        def group(g):
            accs = []
            for j in range(L):
                b = g * L + j
                ts = [urows_v[b, pl.ds(c * L, L)] * irows_v[b, pl.ds(c * L, L)]
                      for c in range(D // L)]
                while len(ts) > 1:
                    ts = [ts[2 * i] + ts[2 * i + 1] for i in range(len(ts) // 2)]
                accs.append(ts[0])
            vecs = accs
            for t in range(4):
                m, key = masks[t], keys[t]
                nxt = []
                for p in range(len(vecs) // 2):
                    a, b2 = vecs[2 * p], vecs[2 * p + 1]
                    x = jnp.where(m, b2, a)
                    y = jnp.where(m, a, b2)
                    _, yx = plsc.sort_key_val(key, y)
                    nxt.append(x + yx)
                vecs = nxt
            out_v[pl.ds(g * L, L)] = vecs[0]

        def step(g, carry):
            for kk in range(nb):
                @pl.when(g == 2 * kk)
                def _(kk=kk):
                    copies[kk][0].wait()
                    copies[kk][1].wait()
            group(g)
            return carry

        lax.fori_loop(0, bpw // L, step, 0)
        pltpu.sync_copy(out_v, out_hbm.at[pl.ds(base, bpw)])

    return scores_kernel


_scores = _build()


def kernel(user_id, user_features, item_id, item_features, position,
           user_table, item_table):
    del user_features, item_features, position
    return _scores(user_id, item_id, user_table, item_table)


# speedup vs baseline: 1.0655x; 1.0080x over previous
import functools

import jax
import jax.numpy as jnp
import numpy as np
from jax import lax
from jax.experimental import pallas as pl
from jax.experimental.pallas import tpu as pltpu
from jax.experimental.pallas import tpu_sc as plsc

BATCH = 4096
D = 128
L = 16


def _build():
    info = plsc.get_sparse_core_info()
    nc, ns = info.num_cores, info.num_subcores
    nw = nc * ns
    bpw = BATCH // nw
    mesh = plsc.VectorSubcoreMesh(core_axis_name="c", subcore_axis_name="s")

    @functools.partial(
        pl.kernel,
        mesh=mesh,
        compiler_params=pltpu.CompilerParams(
            needs_layout_passes=False,
            disable_bounds_checks=True,
            disable_semaphore_checks=True,
            skip_device_barrier=True,
        ),
        out_type=jax.ShapeDtypeStruct((BATCH,), jnp.float32),
        scratch_types=[
            pltpu.VMEM((bpw,), jnp.int32),
            pltpu.VMEM((bpw,), jnp.int32),
            pltpu.VMEM((bpw, D), jnp.float32),
            pltpu.VMEM((bpw, D), jnp.float32),
            pltpu.VMEM((bpw,), jnp.float32),
            [pltpu.SemaphoreType.DMA] * (bpw // L + 1),
        ],
    )
    def scores_kernel(uid_hbm, iid_hbm, ut_hbm, it_hbm, out_hbm,
                      uidx_v, iidx_v, urows_v, irows_v, out_v, sems):
        wid = lax.axis_index("s") * nc + lax.axis_index("c")
        base = wid * bpw
        rpb = L
        nb = bpw // rpb
        cid0 = pltpu.async_copy(uid_hbm.at[pl.ds(base, bpw)], uidx_v, sems[nb])
        cid1 = pltpu.async_copy(iid_hbm.at[pl.ds(base, bpw)], iidx_v, sems[nb])
        cid0.wait()
        cid1.wait()
        copies = []
        for k in range(nb):
            sl = pl.ds(k * rpb, rpb)
            cu = pltpu.async_copy(ut_hbm.at[uidx_v.at[sl]], urows_v.at[sl], sems[k])
            ci = pltpu.async_copy(it_hbm.at[iidx_v.at[sl]], irows_v.at[sl], sems[k])
            copies.append((cu, ci))

        lanes = lax.iota(jnp.int32, L)
        masks = [(lanes & (1 << t)) != 0 for t in range(4)]

        def rot(v, k):
            return jnp.concatenate([v[k:], v[:k]])

        keys = [plsc.bitcast(lanes ^ (1 << t), jnp.uint32) for t in range(4)]

        def group(g):
            accs = []
            for j in range(L):
                b = g * L + j
                ts = [urows_v[b, pl.ds(c * L, L)] * irows_v[b, pl.ds(c * L, L)]
                      for c in range(D // L)]
                while len(ts) > 1:
                    ts = [ts[2 * i] + ts[2 * i + 1] for i in range(len(ts) // 2)]
                accs.append(ts[0])
            vecs = accs
            for t in range(4):
                m, key = masks[t], keys[t]
                nxt = []
                for p in range(len(vecs) // 2):
                    a, b2 = vecs[2 * p], vecs[2 * p + 1]
                    x = jnp.where(m, b2, a)
                    y = jnp.where(m, a, b2)
                    _, yx = plsc.sort_key_val(key, y)
                    nxt.append(x + yx)
                vecs = nxt
            out_v[pl.ds(g * L, L)] = vecs[0]

        gpb = rpb // L
        def step(g, carry):
            for kk in range(nb):
                @pl.when(g == gpb * kk)
                def _(kk=kk):
                    copies[kk][0].wait()
                    copies[kk][1].wait()
            group(g)
            return carry

        lax.fori_loop(0, bpw // L, step, 0)
        pltpu.sync_copy(out_v, out_hbm.at[pl.ds(base, bpw)])

    return scores_kernel


_scores = _build()


def kernel(user_id, user_features, item_id, item_features, position,
           user_table, item_table):
    del user_features, item_features, position
    return _scores(user_id, item_id, user_table, item_table)


# speedup vs baseline: 1.0673x; 1.0017x over previous
import functools

import jax
import jax.numpy as jnp
from jax import lax
from jax.experimental import pallas as pl
from jax.experimental.pallas import tpu as pltpu
from jax.experimental.pallas import tpu_sc as plsc

BATCH = 4096
D = 128
L = 16


def _build():
    info = plsc.get_sparse_core_info()
    nc, ns = info.num_cores, info.num_subcores
    nw = nc * ns
    bpw = BATCH // nw
    mesh = plsc.VectorSubcoreMesh(core_axis_name="c", subcore_axis_name="s")

    @functools.partial(
        pl.kernel,
        mesh=mesh,
        compiler_params=pltpu.CompilerParams(
            needs_layout_passes=False,
            disable_bounds_checks=True,
            disable_semaphore_checks=True,
            skip_device_barrier=True,
        ),
        out_type=jax.ShapeDtypeStruct((BATCH,), jnp.float32),
        scratch_types=[
            pltpu.VMEM((bpw,), jnp.int32),
            pltpu.VMEM((bpw,), jnp.int32),
            pltpu.VMEM((bpw, D), jnp.float32),
            pltpu.VMEM((bpw, D), jnp.float32),
            pltpu.VMEM((bpw,), jnp.float32),
            [pltpu.SemaphoreType.DMA] * (bpw // L + 1),
        ],
    )
    def scores_kernel(uid_hbm, iid_hbm, ut_hbm, it_hbm, out_hbm,
                      uidx_v, iidx_v, urows_v, irows_v, out_v, sems):
        wid = lax.axis_index("s") * nc + lax.axis_index("c")
        base = wid * bpw
        rpb = L
        nb = bpw // rpb
        cid0 = pltpu.async_copy(uid_hbm.at[pl.ds(base, bpw)], uidx_v, sems[nb])
        cid1 = pltpu.async_copy(iid_hbm.at[pl.ds(base, bpw)], iidx_v, sems[nb])
        cid0.wait()
        cid1.wait()
        copies = []
        for k in range(nb):
            sl = pl.ds(k * rpb, rpb)
            cu = pltpu.async_copy(ut_hbm.at[uidx_v.at[sl]], urows_v.at[sl], sems[k])
            ci = pltpu.async_copy(it_hbm.at[iidx_v.at[sl]], irows_v.at[sl], sems[k])
            copies.append((cu, ci))

        lanes = lax.iota(jnp.int32, L)
        masks = [(lanes & (1 << t)) != 0 for t in range(4)]

        keys = [plsc.bitcast(lanes ^ (1 << t), jnp.uint32) for t in range(4)]

        def group(g):
            accs = []
            for j in range(L):
                b = g * L + j
                ts = [urows_v[b, pl.ds(c * L, L)] * irows_v[b, pl.ds(c * L, L)]
                      for c in range(D // L)]
                while len(ts) > 1:
                    ts = [ts[2 * i] + ts[2 * i + 1] for i in range(len(ts) // 2)]
                accs.append(ts[0])
            vecs = accs
            for t in range(4):
                m, key = masks[t], keys[t]
                nxt = []
                for p in range(len(vecs) // 2):
                    a, b2 = vecs[2 * p], vecs[2 * p + 1]
                    x = jnp.where(m, b2, a)
                    y = jnp.where(m, a, b2)
                    _, yx = plsc.sort_key_val(key, y)
                    nxt.append(x + yx)
                vecs = nxt
            out_v[pl.ds(g * L, L)] = vecs[0]

        gpb = rpb // L
        def step(g, carry):
            for kk in range(nb):
                @pl.when(g == gpb * kk)
                def _(kk=kk):
                    copies[kk][0].wait()
                    copies[kk][1].wait()
            group(g)
            return carry

        lax.fori_loop(0, bpw // L, step, 0)
        pltpu.sync_copy(out_v, out_hbm.at[pl.ds(base, bpw)])

    return scores_kernel


_scores = _build()


def kernel(user_id, user_features, item_id, item_features, position,
           user_table, item_table):
    del user_features, item_features, position
    return _scores(user_id, item_id, user_table, item_table)
